# baseline (speedup 1.0000x reference)
import functools

import jax
import jax.numpy as jnp
from jax import lax
from jax.experimental import pallas as pl
from jax.experimental.pallas import tpu as pltpu
from jax.experimental.pallas import tpu_sc as plsc

N = 10000
E = 160000
_BN = 1000
_NTILES = 16
_ROWS_PER_TILE = N // _NTILES


def _leaky(v):
    return jnp.maximum(v, 0.2 * v)


def _softmax_l2(v):
    v = _leaky(v)
    v = v - jnp.max(v, axis=1, keepdims=True)
    e = jnp.exp(v)
    p = e / jnp.sum(e, axis=1, keepdims=True)
    return p / jnp.maximum(jnp.sqrt(jnp.sum(p * p, axis=1, keepdims=True)), 1e-12)


def _mm_body(x_ref, w_ref, o_ref):
    o_ref[...] = jnp.dot(x_ref[...], w_ref[...], preferred_element_type=jnp.float32, precision=lax.Precision.HIGHEST)


def _matmul(x, w):
    n, k = x.shape
    _, m = w.shape
    return pl.pallas_call(
        _mm_body,
        grid=(n // _BN,),
        in_specs=[
            pl.BlockSpec((_BN, k), lambda i: (i, 0)),
            pl.BlockSpec((k, m), lambda i: (0, 0)),
        ],
        out_specs=pl.BlockSpec((_BN, m), lambda i: (i, 0)),
        out_shape=jax.ShapeDtypeStruct((n, m), jnp.float32),
    )(x, w)


def _fuse_mid(z_raw, h, wa, wb, b2, w_next):
    nch = z_raw.shape[0]
    m_out = w_next.shape[2]

    def body(z_ref, h_ref, wa_ref, wb_ref, b_ref, w_ref, o_ref):
        h_ = h_ref[...]
        t = [_leaky(z_ref[c]) for c in range(nch)]
        a = b_ref[...] + jnp.dot(h_, wb_ref[...], preferred_element_type=jnp.float32, precision=lax.Precision.HIGHEST)
        for c in range(nch):
            a = a + jnp.dot(t[c], wa_ref[c], preferred_element_type=jnp.float32, precision=lax.Precision.HIGHEST)
        m = _softmax_l2(a)
        m0, m1 = m[:, 0:1], m[:, 1:2]
        acc = jnp.zeros((_BN, m_out), jnp.float32)
        for c in range(nch):
            f = m0 * t[c] + m1 * h_[:, c * 128:(c + 1) * 128]
            acc = acc + jnp.dot(f, w_ref[c], preferred_element_type=jnp.float32, precision=lax.Precision.HIGHEST)
        o_ref[...] = acc

    return pl.pallas_call(
        body,
        grid=(N // _BN,),
        in_specs=[
            pl.BlockSpec((nch, _BN, 128), lambda i: (0, i, 0)),
            pl.BlockSpec((_BN, 128 * nch), lambda i: (i, 0)),
            pl.BlockSpec(wa.shape, lambda i: (0, 0, 0)),
            pl.BlockSpec(wb.shape, lambda i: (0, 0)),
            pl.BlockSpec((1, 2), lambda i: (0, 0)),
            pl.BlockSpec(w_next.shape, lambda i: (0, 0, 0)),
        ],
        out_specs=pl.BlockSpec((_BN, m_out), lambda i: (i, 0)),
        out_shape=jax.ShapeDtypeStruct((N, m_out), jnp.float32),
    )(z_raw, h, wa, wb, b2, w_next)


def _fuse_final(z1r, z2r, z3r, zz, wl1, wl2, wl3, wlz, blr, wz1, wz2, wz3, wzz):
    def body(z1_ref, z2_ref, z3_ref, z_ref, wl1_ref, wl2_ref, wl3_ref, wlz_ref,
             bl_ref, wz1_ref, wz2_ref, wz3_ref, wzz_ref, o_ref):
        t1 = [_leaky(z1_ref[c]) for c in range(4)]
        t2 = [_leaky(z2_ref[c]) for c in range(2)]
        t3 = [_leaky(z3_ref[c]) for c in range(2)]
        z_ = z_ref[...]
        a = bl_ref[...] + jnp.dot(z_, wlz_ref[...], preferred_element_type=jnp.float32, precision=lax.Precision.HIGHEST)
        for c in range(4):
            a = a + jnp.dot(t1[c], wl1_ref[c], preferred_element_type=jnp.float32, precision=lax.Precision.HIGHEST)
        for c in range(2):
            a = a + jnp.dot(t2[c], wl2_ref[c], preferred_element_type=jnp.float32, precision=lax.Precision.HIGHEST)
            a = a + jnp.dot(t3[c], wl3_ref[c], preferred_element_type=jnp.float32, precision=lax.Precision.HIGHEST)
        u = _softmax_l2(a)
        s1 = jnp.zeros((_BN, 16), jnp.float32)
        for c in range(4):
            s1 = s1 + jnp.dot(t1[c], wz1_ref[c], preferred_element_type=jnp.float32, precision=lax.Precision.HIGHEST)
        s2 = jnp.zeros((_BN, 16), jnp.float32)
        s3 = jnp.zeros((_BN, 16), jnp.float32)
        for c in range(2):
            s2 = s2 + jnp.dot(t2[c], wz2_ref[c], preferred_element_type=jnp.float32, precision=lax.Precision.HIGHEST)
            s3 = s3 + jnp.dot(t3[c], wz3_ref[c], preferred_element_type=jnp.float32, precision=lax.Precision.HIGHEST)
        sz = jnp.dot(z_, wzz_ref[...], preferred_element_type=jnp.float32, precision=lax.Precision.HIGHEST)
        o_ref[...] = (u[:, 0:1] * s1 + u[:, 1:2] * s2 + u[:, 2:3] * s3
                      + u[:, 3:4] * sz)

    return pl.pallas_call(
        body,
        grid=(N // _BN,),
        in_specs=[
            pl.BlockSpec((4, _BN, 128), lambda i: (0, i, 0)),
            pl.BlockSpec((2, _BN, 128), lambda i: (0, i, 0)),
            pl.BlockSpec((2, _BN, 32), lambda i: (0, i, 0)),
            pl.BlockSpec((_BN, 64), lambda i: (i, 0)),
            pl.BlockSpec(wl1.shape, lambda i: (0, 0, 0)),
            pl.BlockSpec(wl2.shape, lambda i: (0, 0, 0)),
            pl.BlockSpec(wl3.shape, lambda i: (0, 0, 0)),
            pl.BlockSpec(wlz.shape, lambda i: (0, 0)),
            pl.BlockSpec((1, 4), lambda i: (0, 0)),
            pl.BlockSpec(wz1.shape, lambda i: (0, 0, 0)),
            pl.BlockSpec(wz2.shape, lambda i: (0, 0, 0)),
            pl.BlockSpec(wz3.shape, lambda i: (0, 0, 0)),
            pl.BlockSpec(wzz.shape, lambda i: (0, 0)),
        ],
        out_specs=pl.BlockSpec((_BN, 16), lambda i: (i, 0)),
        out_shape=jax.ShapeDtypeStruct((N, 16), jnp.float32),
    )(z1r, z2r, z3r, zz, wl1, wl2, wl3, wlz, blr, wz1, wz2, wz3, wzz)


def _finish(p):
    def body(p_ref, o1_ref, o2_ref):
        net = (p_ref[0] + p_ref[1])[:, :10]
        o1_ref[...] = net
        v = net - jnp.max(net, axis=1, keepdims=True)
        e = jnp.exp(v)
        o2_ref[...] = e / jnp.sum(e, axis=1, keepdims=True)

    outs = pl.pallas_call(
        body,
        grid=(N // _BN,),
        in_specs=[pl.BlockSpec((2, _BN, 16), lambda i: (0, i, 0))],
        out_specs=[
            pl.BlockSpec((_BN, 10), lambda i: (i, 0)),
            pl.BlockSpec((_BN, 10), lambda i: (i, 0)),
        ],
        out_shape=[
            jax.ShapeDtypeStruct((N, 10), jnp.float32),
            jax.ShapeDtypeStruct((N, 10), jnp.float32),
        ],
    )(p)
    return outs[0], outs[1]


def _make_spmm(nch, C, rounds, split_edges):
    ntasks = _NTILES * (2 if split_edges else 1)
    e_per_tile = E // ntasks
    chunk = 80 if nch > 1 else 125
    iters = e_per_tile // chunk
    nbuf = 3 if C == 128 else 6
    n_out = 2 if split_edges else 2 * rounds
    rmain = (N // _NTILES) // 8 * 8
    rtail = N - (_NTILES - 1) * rmain
    mesh = plsc.VectorSubcoreMesh(core_axis_name="c", subcore_axis_name="s")

    @functools.partial(
        pl.kernel,
        out_type=jax.ShapeDtypeStruct((n_out, N, C), jnp.float32),
        mesh=mesh,
        compiler_params=pltpu.CompilerParams(use_tc_tiling_on_sc=False),
        scratch_types=(
            [
                pltpu.VMEM((iters, chunk), jnp.int32),
                pltpu.VMEM((iters, chunk), jnp.int32),
            ]
            + [pltpu.VMEM((chunk,), jnp.int32) for _ in range(nbuf)]
            + [pltpu.VMEM((chunk, C), jnp.float32) for _ in range(nbuf)]
            + [pltpu.VMEM_SHARED((N, C), jnp.float32)]
            + [pltpu.SemaphoreType.DMA for _ in range(2 * nbuf)]
        ),
    )
    def spmm(sup_hbm, src_hbm, dst_hbm, zeros_hbm, out_hbm, src_all, dst_all,
             *bufs):
        srcx = bufs[:nbuf]
        rows = bufs[nbuf:2 * nbuf]
        acc = bufs[2 * nbuf]
        semg = bufs[2 * nbuf + 1:3 * nbuf + 1]
        sems = bufs[3 * nbuf + 1:]
        core = lax.axis_index("c")
        sid = lax.axis_index("s")
        tid = core * _NTILES + sid if split_edges else sid
        pltpu.sync_copy(src_hbm.at[tid], src_all)
        pltpu.sync_copy(dst_hbm.at[tid], dst_all)
        rbase = sid * rmain
        last = sid == _NTILES - 1

        for r in range(rounds):
            ch = 2 * r + core

            def gather_start(i, p):
                if nch > 1:
                    for j in range(chunk // 16):
                        sl = pl.ds(j * 16, 16)
                        srcx[p][sl] = src_all[i, sl] * nch + ch
                    pltpu.async_copy(sup_hbm.at[srcx[p]], rows[p], semg[p])
                else:
                    pltpu.async_copy(sup_hbm.at[src_all.at[i]], rows[p], semg[p])

            def gather_wait(i, p):
                idx = srcx[p] if nch > 1 else src_all.at[i]
                pltpu.make_async_copy(sup_hbm.at[idx], rows[p], semg[p]).wait()

            def scatter_start(i, p):
                pltpu.async_copy(rows[p], acc.at[dst_all.at[i]], sems[p], add=True)

            def scatter_wait(i, p):
                pltpu.make_async_copy(rows[p], acc.at[dst_all.at[i]], sems[p]).wait()

            for p in range(nbuf):
                gather_start(p, p)
            pltpu.sync_copy(zeros_hbm.at[pl.ds(rbase, rmain)],
                            acc.at[pl.ds(rbase, rmain)])

            @pl.when(last)
            def _():
                pltpu.sync_copy(zeros_hbm.at[pl.ds(rmain * _NTILES, rtail - rmain)],
                                acc.at[pl.ds(rmain * _NTILES, rtail - rmain)])

            plsc.subcore_barrier()

            def body(blk, carry):
                for b in range(nbuf):
                    i = blk * nbuf + b
                    gather_wait(i, b)
                    scatter_start(i, b)
                    k = i + nbuf - 1
                    pb = (b - 1) % nbuf

                    @pl.when((i >= 1) & (k < iters))
                    def _():
                        scatter_wait(i - 1, pb)
                        gather_start(k, pb)

                return carry

            lax.fori_loop(0, iters // nbuf, body, 0)
            for i in range((iters // nbuf) * nbuf, iters):
                gather_wait(i, i % nbuf)
                scatter_start(i, i % nbuf)
            for j in range(max(iters - nbuf, 0), iters):
                scatter_wait(j, j % nbuf)
            plsc.subcore_barrier()
            oc = core if split_edges else ch
            pltpu.sync_copy(acc.at[pl.ds(rbase, rmain)],
                            out_hbm.at[oc, pl.ds(rbase, rmain)])

            @pl.when(last)
            def _():
                pltpu.sync_copy(acc.at[pl.ds(rmain * _NTILES, rtail - rmain)],
                                out_hbm.at[oc, pl.ds(rmain * _NTILES, rtail - rmain)])

    return spmm


_SPMM1 = _make_spmm(4, 128, 2, False)
_SPMM2 = _make_spmm(2, 128, 1, False)
_SPMM3 = _make_spmm(2, 32, 1, False)
_SPMM4 = _make_spmm(1, 16, 1, True)


def kernel(x, h1, h2, z, edge_index, W0, W1, W2, Wz, Wl, bl, Wm1, bm1, Wm2, bm2):
    dst = edge_index[0]
    src = edge_index[1]
    src80 = src.reshape(16, E // (16 * 80), 80)
    dst80 = dst.reshape(16, E // (16 * 80), 80)
    src40 = src.reshape(32, E // (32 * 125), 125)
    dst40 = dst.reshape(32, E // (32 * 125), 125)
    z128 = jnp.zeros((N, 128), jnp.float32)
    z32 = jnp.zeros((N, 32), jnp.float32)
    z16 = jnp.zeros((N, 16), jnp.float32)

    sup1 = _matmul(x, W0)
    z1r = _SPMM1(sup1.reshape(N * 4, 128), src80, dst80, z128)

    sup2 = _fuse_mid(z1r, h1, Wm1[:512].reshape(4, 128, 2), Wm1[512:],
                     bm1.reshape(1, 2), W1.reshape(4, 128, 256))
    z2r = _SPMM2(sup2.reshape(N * 2, 128), src80, dst80, z128)

    sup3 = _fuse_mid(z2r, h2, Wm2[:256].reshape(2, 128, 2), Wm2[256:],
                     bm2.reshape(1, 2), W2.reshape(2, 128, 64))
    z3r = _SPMM3(sup3.reshape(N * 2, 32), src80, dst80, z32)

    Wzp = jnp.pad(Wz, ((0, 0), (0, 6)))
    sup4 = _fuse_final(
        z1r, z2r, z3r, z,
        Wl[:512].reshape(4, 128, 4), Wl[512:768].reshape(2, 128, 4),
        Wl[768:832].reshape(2, 32, 4), Wl[832:], bl.reshape(1, 4),
        Wzp[:512].reshape(4, 128, 16), Wzp[512:768].reshape(2, 128, 16),
        Wzp[768:832].reshape(2, 32, 16), Wzp[832:])
    p = _SPMM4(sup4, src40, dst40, z16)
    return _finish(p)


# speedup vs baseline: 1.5616x; 1.5616x over previous
import functools

import jax
import jax.numpy as jnp
from jax import lax
from jax.experimental import pallas as pl
from jax.experimental.pallas import tpu as pltpu
from jax.experimental.pallas import tpu_sc as plsc

N = 10000
E = 160000
_BN = 1000
_NTILES = 16
_ROWS_PER_TILE = N // _NTILES


def _leaky(v):
    return jnp.maximum(v, 0.2 * v)


def _softmax_l2(v):
    v = _leaky(v)
    v = v - jnp.max(v, axis=1, keepdims=True)
    e = jnp.exp(v)
    p = e / jnp.sum(e, axis=1, keepdims=True)
    return p / jnp.maximum(jnp.sqrt(jnp.sum(p * p, axis=1, keepdims=True)), 1e-12)


def _mm_body(x_ref, w_ref, o_ref):
    o_ref[...] = jnp.dot(x_ref[...], w_ref[...], preferred_element_type=jnp.float32)


def _matmul(x, w):
    n, k = x.shape
    _, m = w.shape
    return pl.pallas_call(
        _mm_body,
        grid=(n // _BN,),
        in_specs=[
            pl.BlockSpec((_BN, k), lambda i: (i, 0)),
            pl.BlockSpec((k, m), lambda i: (0, 0)),
        ],
        out_specs=pl.BlockSpec((_BN, m), lambda i: (i, 0)),
        out_shape=jax.ShapeDtypeStruct((n, m), jnp.float32),
    )(x, w)


def _fuse_mid(z_raw, h, wa, wb, b2, w_next):
    nch = z_raw.shape[0]
    m_out = w_next.shape[2]

    def body(z_ref, h_ref, wa_ref, wb_ref, b_ref, w_ref, o_ref):
        h_ = h_ref[...]
        t = [_leaky(z_ref[c]) for c in range(nch)]
        a = b_ref[...] + jnp.dot(h_, wb_ref[...], preferred_element_type=jnp.float32)
        for c in range(nch):
            a = a + jnp.dot(t[c], wa_ref[c], preferred_element_type=jnp.float32)
        m = _softmax_l2(a)
        m0, m1 = m[:, 0:1], m[:, 1:2]
        acc = jnp.zeros((_BN, m_out), jnp.float32)
        for c in range(nch):
            f = m0 * t[c] + m1 * h_[:, c * 128:(c + 1) * 128]
            acc = acc + jnp.dot(f, w_ref[c], preferred_element_type=jnp.float32)
        o_ref[...] = acc

    return pl.pallas_call(
        body,
        grid=(N // _BN,),
        in_specs=[
            pl.BlockSpec((nch, _BN, 128), lambda i: (0, i, 0)),
            pl.BlockSpec((_BN, 128 * nch), lambda i: (i, 0)),
            pl.BlockSpec(wa.shape, lambda i: (0, 0, 0)),
            pl.BlockSpec(wb.shape, lambda i: (0, 0)),
            pl.BlockSpec((1, 2), lambda i: (0, 0)),
            pl.BlockSpec(w_next.shape, lambda i: (0, 0, 0)),
        ],
        out_specs=pl.BlockSpec((_BN, m_out), lambda i: (i, 0)),
        out_shape=jax.ShapeDtypeStruct((N, m_out), jnp.float32),
    )(z_raw, h, wa, wb, b2, w_next)


def _fuse_final(z1r, z2r, z3r, zz, wl1, wl2, wl3, wlz, blr, wz1, wz2, wz3, wzz):
    def body(z1_ref, z2_ref, z3_ref, z_ref, wl1_ref, wl2_ref, wl3_ref, wlz_ref,
             bl_ref, wz1_ref, wz2_ref, wz3_ref, wzz_ref, o_ref):
        t1 = [_leaky(z1_ref[c]) for c in range(4)]
        t2 = [_leaky(z2_ref[c]) for c in range(2)]
        t3 = [_leaky(z3_ref[c]) for c in range(2)]
        z_ = z_ref[...]
        a = bl_ref[...] + jnp.dot(z_, wlz_ref[...], preferred_element_type=jnp.float32)
        for c in range(4):
            a = a + jnp.dot(t1[c], wl1_ref[c], preferred_element_type=jnp.float32)
        for c in range(2):
            a = a + jnp.dot(t2[c], wl2_ref[c], preferred_element_type=jnp.float32)
            a = a + jnp.dot(t3[c], wl3_ref[c], preferred_element_type=jnp.float32)
        u = _softmax_l2(a)
        s1 = jnp.zeros((_BN, 16), jnp.float32)
        for c in range(4):
            s1 = s1 + jnp.dot(t1[c], wz1_ref[c], preferred_element_type=jnp.float32)
        s2 = jnp.zeros((_BN, 16), jnp.float32)
        s3 = jnp.zeros((_BN, 16), jnp.float32)
        for c in range(2):
            s2 = s2 + jnp.dot(t2[c], wz2_ref[c], preferred_element_type=jnp.float32)
            s3 = s3 + jnp.dot(t3[c], wz3_ref[c], preferred_element_type=jnp.float32)
        sz = jnp.dot(z_, wzz_ref[...], preferred_element_type=jnp.float32)
        o_ref[...] = (u[:, 0:1] * s1 + u[:, 1:2] * s2 + u[:, 2:3] * s3
                      + u[:, 3:4] * sz)

    return pl.pallas_call(
        body,
        grid=(N // _BN,),
        in_specs=[
            pl.BlockSpec((4, _BN, 128), lambda i: (0, i, 0)),
            pl.BlockSpec((2, _BN, 128), lambda i: (0, i, 0)),
            pl.BlockSpec((2, _BN, 32), lambda i: (0, i, 0)),
            pl.BlockSpec((_BN, 64), lambda i: (i, 0)),
            pl.BlockSpec(wl1.shape, lambda i: (0, 0, 0)),
            pl.BlockSpec(wl2.shape, lambda i: (0, 0, 0)),
            pl.BlockSpec(wl3.shape, lambda i: (0, 0, 0)),
            pl.BlockSpec(wlz.shape, lambda i: (0, 0)),
            pl.BlockSpec((1, 4), lambda i: (0, 0)),
            pl.BlockSpec(wz1.shape, lambda i: (0, 0, 0)),
            pl.BlockSpec(wz2.shape, lambda i: (0, 0, 0)),
            pl.BlockSpec(wz3.shape, lambda i: (0, 0, 0)),
            pl.BlockSpec(wzz.shape, lambda i: (0, 0)),
        ],
        out_specs=pl.BlockSpec((_BN, 16), lambda i: (i, 0)),
        out_shape=jax.ShapeDtypeStruct((N, 16), jnp.float32),
    )(z1r, z2r, z3r, zz, wl1, wl2, wl3, wlz, blr, wz1, wz2, wz3, wzz)


def _finish(p):
    def body(p_ref, o1_ref, o2_ref):
        net = (p_ref[0] + p_ref[1])[:, :10]
        o1_ref[...] = net
        v = net - jnp.max(net, axis=1, keepdims=True)
        e = jnp.exp(v)
        o2_ref[...] = e / jnp.sum(e, axis=1, keepdims=True)

    outs = pl.pallas_call(
        body,
        grid=(N // _BN,),
        in_specs=[pl.BlockSpec((2, _BN, 16), lambda i: (0, i, 0))],
        out_specs=[
            pl.BlockSpec((_BN, 10), lambda i: (i, 0)),
            pl.BlockSpec((_BN, 10), lambda i: (i, 0)),
        ],
        out_shape=[
            jax.ShapeDtypeStruct((N, 10), jnp.float32),
            jax.ShapeDtypeStruct((N, 10), jnp.float32),
        ],
    )(p)
    return outs[0], outs[1]


def _make_spmm(nch, C, rounds, split_edges):
    ntasks = _NTILES * (2 if split_edges else 1)
    e_per_tile = E // ntasks
    chunk = 80 if nch > 1 else 125
    iters = e_per_tile // chunk
    nbuf = 3 if C == 128 else 6
    n_out = 2 if split_edges else 2 * rounds
    rmain = (N // _NTILES) // 8 * 8
    rtail = N - (_NTILES - 1) * rmain
    mesh = plsc.VectorSubcoreMesh(core_axis_name="c", subcore_axis_name="s")

    @functools.partial(
        pl.kernel,
        out_type=jax.ShapeDtypeStruct((n_out, N, C), jnp.float32),
        mesh=mesh,
        compiler_params=pltpu.CompilerParams(use_tc_tiling_on_sc=False),
        scratch_types=(
            [
                pltpu.VMEM((iters, chunk), jnp.int32),
                pltpu.VMEM((iters, chunk), jnp.int32),
            ]
            + [pltpu.VMEM((chunk,), jnp.int32) for _ in range(nbuf)]
            + [pltpu.VMEM((chunk, C), jnp.float32) for _ in range(nbuf)]
            + [pltpu.VMEM_SHARED((N, C), jnp.float32)]
            + [pltpu.SemaphoreType.DMA for _ in range(nbuf)]
        ),
    )
    def spmm(sup_hbm, src_hbm, dst_hbm, zeros_hbm, out_hbm, src_all, dst_all,
             *bufs):
        srcx = bufs[:nbuf]
        rows = bufs[nbuf:2 * nbuf]
        acc = bufs[2 * nbuf]
        semg = bufs[2 * nbuf + 1:]
        core = lax.axis_index("c")
        sid = lax.axis_index("s")
        tid = core * _NTILES + sid if split_edges else sid
        pltpu.sync_copy(src_hbm.at[tid], src_all)
        pltpu.sync_copy(dst_hbm.at[tid], dst_all)
        rbase = sid * rmain
        last = sid == _NTILES - 1

        for r in range(rounds):
            ch = 2 * r + core

            def gather_start(i, p):
                if nch > 1:
                    for j in range(chunk // 16):
                        sl = pl.ds(j * 16, 16)
                        srcx[p][sl] = src_all[i, sl] * nch + ch
                    pltpu.async_copy(sup_hbm.at[srcx[p]], rows[p], semg[p])
                else:
                    pltpu.async_copy(sup_hbm.at[src_all.at[i]], rows[p], semg[p])

            def gather_wait(i, p):
                idx = srcx[p] if nch > 1 else src_all.at[i]
                pltpu.make_async_copy(sup_hbm.at[idx], rows[p], semg[p]).wait()

            def scatter(i, p):
                pltpu.sync_copy(rows[p], acc.at[dst_all.at[i]], add=True)

            for p in range(nbuf):
                gather_start(p, p)
            pltpu.sync_copy(zeros_hbm.at[pl.ds(rbase, rmain)],
                            acc.at[pl.ds(rbase, rmain)])

            @pl.when(last)
            def _():
                pltpu.sync_copy(zeros_hbm.at[pl.ds(rmain * _NTILES, rtail - rmain)],
                                acc.at[pl.ds(rmain * _NTILES, rtail - rmain)])

            plsc.subcore_barrier()

            def body(blk, carry):
                for b in range(nbuf):
                    i = blk * nbuf + b
                    gather_wait(i, b)
                    k = i + nbuf - 1
                    pb = (b - 1) % nbuf

                    @pl.when((i >= 1) & (k < iters))
                    def _():
                        gather_start(k, pb)

                    scatter(i, b)

                return carry

            lax.fori_loop(0, iters // nbuf, body, 0)
            for i in range((iters // nbuf) * nbuf, iters):
                gather_wait(i, i % nbuf)
                scatter(i, i % nbuf)
            plsc.subcore_barrier()
            oc = core if split_edges else ch
            pltpu.sync_copy(acc.at[pl.ds(rbase, rmain)],
                            out_hbm.at[oc, pl.ds(rbase, rmain)])

            @pl.when(last)
            def _():
                pltpu.sync_copy(acc.at[pl.ds(rmain * _NTILES, rtail - rmain)],
                                out_hbm.at[oc, pl.ds(rmain * _NTILES, rtail - rmain)])

    return spmm


_SPMM1 = _make_spmm(4, 128, 2, False)
_SPMM2 = _make_spmm(2, 128, 1, False)
_SPMM3 = _make_spmm(2, 32, 1, False)
_SPMM4 = _make_spmm(1, 16, 1, True)


def kernel(x, h1, h2, z, edge_index, W0, W1, W2, Wz, Wl, bl, Wm1, bm1, Wm2, bm2):
    dst = edge_index[0]
    src = edge_index[1]
    src80 = src.reshape(16, E // (16 * 80), 80)
    dst80 = dst.reshape(16, E // (16 * 80), 80)
    src40 = src.reshape(32, E // (32 * 125), 125)
    dst40 = dst.reshape(32, E // (32 * 125), 125)
    z128 = jnp.zeros((N, 128), jnp.float32)
    z32 = jnp.zeros((N, 32), jnp.float32)
    z16 = jnp.zeros((N, 16), jnp.float32)

    sup1 = _matmul(x, W0)
    z1r = _SPMM1(sup1.reshape(N * 4, 128), src80, dst80, z128)

    sup2 = _fuse_mid(z1r, h1, Wm1[:512].reshape(4, 128, 2), Wm1[512:],
                     bm1.reshape(1, 2), W1.reshape(4, 128, 256))
    z2r = _SPMM2(sup2.reshape(N * 2, 128), src80, dst80, z128)

    sup3 = _fuse_mid(z2r, h2, Wm2[:256].reshape(2, 128, 2), Wm2[256:],
                     bm2.reshape(1, 2), W2.reshape(2, 128, 64))
    z3r = _SPMM3(sup3.reshape(N * 2, 32), src80, dst80, z32)

    Wzp = jnp.pad(Wz, ((0, 0), (0, 6)))
    sup4 = _fuse_final(
        z1r, z2r, z3r, z,
        Wl[:512].reshape(4, 128, 4), Wl[512:768].reshape(2, 128, 4),
        Wl[768:832].reshape(2, 32, 4), Wl[832:], bl.reshape(1, 4),
        Wzp[:512].reshape(4, 128, 16), Wzp[512:768].reshape(2, 128, 16),
        Wzp[768:832].reshape(2, 32, 16), Wzp[832:])
    p = _SPMM4(sup4, src40, dst40, z16)
    return _finish(p)


# speedup vs baseline: 1.6022x; 1.0260x over previous
import functools

import jax
import jax.numpy as jnp
from jax import lax
from jax.experimental import pallas as pl
from jax.experimental.pallas import tpu as pltpu
from jax.experimental.pallas import tpu_sc as plsc

N = 10000
E = 160000
_BN = 1000
_NTILES = 16
_ROWS_PER_TILE = N // _NTILES


def _leaky(v):
    return jnp.maximum(v, 0.2 * v)


def _softmax_l2(v):
    v = _leaky(v)
    v = v - jnp.max(v, axis=1, keepdims=True)
    e = jnp.exp(v)
    p = e / jnp.sum(e, axis=1, keepdims=True)
    return p / jnp.maximum(jnp.sqrt(jnp.sum(p * p, axis=1, keepdims=True)), 1e-12)


def _matmul(x, w):
    n, k = x.shape
    _, m = w.shape
    nch = m // 128

    def body(x_ref, w_ref, o_ref):
        acc = jnp.dot(x_ref[...], w_ref[...], preferred_element_type=jnp.float32)
        for c in range(nch):
            o_ref[c] = acc[:, c * 128:(c + 1) * 128]

    return pl.pallas_call(
        body,
        grid=(n // _BN,),
        in_specs=[
            pl.BlockSpec((_BN, k), lambda i: (i, 0)),
            pl.BlockSpec((k, m), lambda i: (0, 0)),
        ],
        out_specs=pl.BlockSpec((nch, _BN, 128), lambda i: (0, i, 0)),
        out_shape=jax.ShapeDtypeStruct((nch, n, 128), jnp.float32),
    )(x, w)


def _fuse_mid(z_raw, h, wa, wb, b2, w_next, out_c):
    nch = z_raw.shape[0]
    m_out = w_next.shape[2]
    nout = m_out // out_c

    def body(z_ref, h_ref, wa_ref, wb_ref, b_ref, w_ref, o_ref):
        h_ = h_ref[...]
        t = [_leaky(z_ref[c]) for c in range(nch)]
        a = b_ref[...] + jnp.dot(h_, wb_ref[...], preferred_element_type=jnp.float32)
        for c in range(nch):
            a = a + jnp.dot(t[c], wa_ref[c], preferred_element_type=jnp.float32)
        m = _softmax_l2(a)
        m0, m1 = m[:, 0:1], m[:, 1:2]
        acc = jnp.zeros((_BN, m_out), jnp.float32)
        for c in range(nch):
            f = m0 * t[c] + m1 * h_[:, c * 128:(c + 1) * 128]
            acc = acc + jnp.dot(f, w_ref[c], preferred_element_type=jnp.float32)
        for c in range(nout):
            o_ref[c] = acc[:, c * out_c:(c + 1) * out_c]

    return pl.pallas_call(
        body,
        grid=(N // _BN,),
        in_specs=[
            pl.BlockSpec((nch, _BN, 128), lambda i: (0, i, 0)),
            pl.BlockSpec((_BN, 128 * nch), lambda i: (i, 0)),
            pl.BlockSpec(wa.shape, lambda i: (0, 0, 0)),
            pl.BlockSpec(wb.shape, lambda i: (0, 0)),
            pl.BlockSpec((1, 2), lambda i: (0, 0)),
            pl.BlockSpec(w_next.shape, lambda i: (0, 0, 0)),
        ],
        out_specs=pl.BlockSpec((nout, _BN, out_c), lambda i: (0, i, 0)),
        out_shape=jax.ShapeDtypeStruct((nout, N, out_c), jnp.float32),
    )(z_raw, h, wa, wb, b2, w_next)


def _fuse_final(z1r, z2r, z3r, zz, wl1, wl2, wl3, wlz, blr, wz1, wz2, wz3, wzz):
    def body(z1_ref, z2_ref, z3_ref, z_ref, wl1_ref, wl2_ref, wl3_ref, wlz_ref,
             bl_ref, wz1_ref, wz2_ref, wz3_ref, wzz_ref, o_ref):
        t1 = [_leaky(z1_ref[c]) for c in range(4)]
        t2 = [_leaky(z2_ref[c]) for c in range(2)]
        t3 = [_leaky(z3_ref[c]) for c in range(2)]
        z_ = z_ref[...]
        a = bl_ref[...] + jnp.dot(z_, wlz_ref[...], preferred_element_type=jnp.float32)
        for c in range(4):
            a = a + jnp.dot(t1[c], wl1_ref[c], preferred_element_type=jnp.float32)
        for c in range(2):
            a = a + jnp.dot(t2[c], wl2_ref[c], preferred_element_type=jnp.float32)
            a = a + jnp.dot(t3[c], wl3_ref[c], preferred_element_type=jnp.float32)
        u = _softmax_l2(a)
        s1 = jnp.zeros((_BN, 16), jnp.float32)
        for c in range(4):
            s1 = s1 + jnp.dot(t1[c], wz1_ref[c], preferred_element_type=jnp.float32)
        s2 = jnp.zeros((_BN, 16), jnp.float32)
        s3 = jnp.zeros((_BN, 16), jnp.float32)
        for c in range(2):
            s2 = s2 + jnp.dot(t2[c], wz2_ref[c], preferred_element_type=jnp.float32)
            s3 = s3 + jnp.dot(t3[c], wz3_ref[c], preferred_element_type=jnp.float32)
        sz = jnp.dot(z_, wzz_ref[...], preferred_element_type=jnp.float32)
        o_ref[...] = (u[:, 0:1] * s1 + u[:, 1:2] * s2 + u[:, 2:3] * s3
                      + u[:, 3:4] * sz)

    return pl.pallas_call(
        body,
        grid=(N // _BN,),
        in_specs=[
            pl.BlockSpec((4, _BN, 128), lambda i: (0, i, 0)),
            pl.BlockSpec((2, _BN, 128), lambda i: (0, i, 0)),
            pl.BlockSpec((2, _BN, 32), lambda i: (0, i, 0)),
            pl.BlockSpec((_BN, 64), lambda i: (i, 0)),
            pl.BlockSpec(wl1.shape, lambda i: (0, 0, 0)),
            pl.BlockSpec(wl2.shape, lambda i: (0, 0, 0)),
            pl.BlockSpec(wl3.shape, lambda i: (0, 0, 0)),
            pl.BlockSpec(wlz.shape, lambda i: (0, 0)),
            pl.BlockSpec((1, 4), lambda i: (0, 0)),
            pl.BlockSpec(wz1.shape, lambda i: (0, 0, 0)),
            pl.BlockSpec(wz2.shape, lambda i: (0, 0, 0)),
            pl.BlockSpec(wz3.shape, lambda i: (0, 0, 0)),
            pl.BlockSpec(wzz.shape, lambda i: (0, 0)),
        ],
        out_specs=pl.BlockSpec((_BN, 16), lambda i: (i, 0)),
        out_shape=jax.ShapeDtypeStruct((N, 16), jnp.float32),
    )(z1r, z2r, z3r, zz, wl1, wl2, wl3, wlz, blr, wz1, wz2, wz3, wzz)


def _finish(p):
    def body(p_ref, o1_ref, o2_ref):
        net = (p_ref[0] + p_ref[1])[:, :10]
        o1_ref[...] = net
        v = net - jnp.max(net, axis=1, keepdims=True)
        e = jnp.exp(v)
        o2_ref[...] = e / jnp.sum(e, axis=1, keepdims=True)

    outs = pl.pallas_call(
        body,
        grid=(N // _BN,),
        in_specs=[pl.BlockSpec((2, _BN, 16), lambda i: (0, i, 0))],
        out_specs=[
            pl.BlockSpec((_BN, 10), lambda i: (i, 0)),
            pl.BlockSpec((_BN, 10), lambda i: (i, 0)),
        ],
        out_shape=[
            jax.ShapeDtypeStruct((N, 10), jnp.float32),
            jax.ShapeDtypeStruct((N, 10), jnp.float32),
        ],
    )(p)
    return outs[0], outs[1]


def _make_spmm(nch, C, rounds, split_edges):
    ntasks = _NTILES * (2 if split_edges else 1)
    e_per_tile = E // ntasks
    chunk = 80 if nch > 1 else 125
    iters = e_per_tile // chunk
    nbuf = 3 if C == 128 else 6
    n_out = 2 if split_edges else 2 * rounds
    rmain = (N // _NTILES) // 8 * 8
    rtail = N - (_NTILES - 1) * rmain
    mesh = plsc.VectorSubcoreMesh(core_axis_name="c", subcore_axis_name="s")

    @functools.partial(
        pl.kernel,
        out_type=jax.ShapeDtypeStruct((n_out, N, C), jnp.float32),
        mesh=mesh,
        compiler_params=pltpu.CompilerParams(use_tc_tiling_on_sc=False),
        scratch_types=(
            [
                pltpu.VMEM((iters, chunk), jnp.int32),
                pltpu.VMEM((iters, chunk), jnp.int32),
            ]
            + [pltpu.VMEM((chunk,), jnp.int32) for _ in range(nbuf)]
            + [pltpu.VMEM((chunk, C), jnp.float32) for _ in range(nbuf)]
            + [pltpu.VMEM_SHARED((N, C), jnp.float32)]
            + [pltpu.SemaphoreType.DMA for _ in range(nbuf)]
        ),
    )
    def spmm(sup_hbm, src_hbm, dst_hbm, zeros_hbm, out_hbm, src_all, dst_all,
             *bufs):
        srcx = bufs[:nbuf]
        rows = bufs[nbuf:2 * nbuf]
        acc = bufs[2 * nbuf]
        semg = bufs[2 * nbuf + 1:]
        core = lax.axis_index("c")
        sid = lax.axis_index("s")
        tid = core * _NTILES + sid if split_edges else sid
        pltpu.sync_copy(src_hbm.at[tid], src_all)
        pltpu.sync_copy(dst_hbm.at[tid], dst_all)
        rbase = sid * rmain
        last = sid == _NTILES - 1

        for r in range(rounds):
            ch = 2 * r + core
            row_off = ch * N

            def gather_start(i, p):
                if nch > 1:
                    for j in range(chunk // 16):
                        sl = pl.ds(j * 16, 16)
                        srcx[p][sl] = src_all[i, sl] + row_off
                    pltpu.async_copy(sup_hbm.at[srcx[p]], rows[p], semg[p])
                else:
                    pltpu.async_copy(sup_hbm.at[src_all.at[i]], rows[p], semg[p])

            def gather_wait(i, p):
                idx = srcx[p] if nch > 1 else src_all.at[i]
                pltpu.make_async_copy(sup_hbm.at[idx], rows[p], semg[p]).wait()

            def scatter(i, p):
                pltpu.sync_copy(rows[p], acc.at[dst_all.at[i]], add=True)

            for p in range(nbuf):
                gather_start(p, p)
            pltpu.sync_copy(zeros_hbm.at[pl.ds(rbase, rmain)],
                            acc.at[pl.ds(rbase, rmain)])

            @pl.when(last)
            def _():
                pltpu.sync_copy(zeros_hbm.at[pl.ds(rmain * _NTILES, rtail - rmain)],
                                acc.at[pl.ds(rmain * _NTILES, rtail - rmain)])

            plsc.subcore_barrier()

            def body(blk, carry):
                for b in range(nbuf):
                    i = blk * nbuf + b
                    gather_wait(i, b)
                    k = i + nbuf - 1
                    pb = (b - 1) % nbuf

                    @pl.when((i >= 1) & (k < iters))
                    def _():
                        gather_start(k, pb)

                    scatter(i, b)

                return carry

            lax.fori_loop(0, iters // nbuf, body, 0)
            for i in range((iters // nbuf) * nbuf, iters):
                gather_wait(i, i % nbuf)
                scatter(i, i % nbuf)
            plsc.subcore_barrier()
            oc = core if split_edges else ch
            pltpu.sync_copy(acc.at[pl.ds(rbase, rmain)],
                            out_hbm.at[oc, pl.ds(rbase, rmain)])

            @pl.when(last)
            def _():
                pltpu.sync_copy(acc.at[pl.ds(rmain * _NTILES, rtail - rmain)],
                                out_hbm.at[oc, pl.ds(rmain * _NTILES, rtail - rmain)])

    return spmm


_SPMM1 = _make_spmm(4, 128, 2, False)
_SPMM2 = _make_spmm(2, 128, 1, False)
_SPMM3 = _make_spmm(2, 32, 1, False)
_SPMM4 = _make_spmm(1, 16, 1, True)


def kernel(x, h1, h2, z, edge_index, W0, W1, W2, Wz, Wl, bl, Wm1, bm1, Wm2, bm2):
    dst = edge_index[0]
    src = edge_index[1]
    src80 = src.reshape(16, E // (16 * 80), 80)
    dst80 = dst.reshape(16, E // (16 * 80), 80)
    src40 = src.reshape(32, E // (32 * 125), 125)
    dst40 = dst.reshape(32, E // (32 * 125), 125)
    z128 = jnp.zeros((N, 128), jnp.float32)
    z32 = jnp.zeros((N, 32), jnp.float32)
    z16 = jnp.zeros((N, 16), jnp.float32)

    sup1 = _matmul(x, W0)
    z1r = _SPMM1(sup1.reshape(N * 4, 128), src80, dst80, z128)

    sup2 = _fuse_mid(z1r, h1, Wm1[:512].reshape(4, 128, 2), Wm1[512:],
                     bm1.reshape(1, 2), W1.reshape(4, 128, 256), 128)
    z2r = _SPMM2(sup2.reshape(N * 2, 128), src80, dst80, z128)

    sup3 = _fuse_mid(z2r, h2, Wm2[:256].reshape(2, 128, 2), Wm2[256:],
                     bm2.reshape(1, 2), W2.reshape(2, 128, 64), 32)
    z3r = _SPMM3(sup3.reshape(N * 2, 32), src80, dst80, z32)

    Wzp = jnp.pad(Wz, ((0, 0), (0, 6)))
    sup4 = _fuse_final(
        z1r, z2r, z3r, z,
        Wl[:512].reshape(4, 128, 4), Wl[512:768].reshape(2, 128, 4),
        Wl[768:832].reshape(2, 32, 4), Wl[832:], bl.reshape(1, 4),
        Wzp[:512].reshape(4, 128, 16), Wzp[512:768].reshape(2, 128, 16),
        Wzp[768:832].reshape(2, 32, 16), Wzp[832:])
    p = _SPMM4(sup4, src40, dst40, z16)
    return _finish(p)


# speedup vs baseline: 1.7467x; 1.0902x over previous
import functools

import jax
import jax.numpy as jnp
from jax import lax
from jax.experimental import pallas as pl
from jax.experimental.pallas import tpu as pltpu
from jax.experimental.pallas import tpu_sc as plsc

N = 10000
E = 160000
_BN = 2000
_NTILES = 16
_ROWS_PER_TILE = N // _NTILES


def _leaky(v):
    return jnp.maximum(v, 0.2 * v)


def _softmax_l2(v):
    v = _leaky(v)
    e = jnp.exp(v - jnp.max(v, axis=1, keepdims=True))
    return e / jnp.sqrt(jnp.sum(e * e, axis=1, keepdims=True))


def _matmul(x, w):
    n, k = x.shape
    _, m = w.shape
    nch = m // 128

    def body(x_ref, w_ref, o_ref):
        acc = jnp.dot(x_ref[...], w_ref[...], preferred_element_type=jnp.float32)
        for c in range(nch):
            o_ref[c] = acc[:, c * 128:(c + 1) * 128]

    return pl.pallas_call(
        body,
        grid=(n // _BN,),
        in_specs=[
            pl.BlockSpec((_BN, k), lambda i: (i, 0)),
            pl.BlockSpec((k, m), lambda i: (0, 0)),
        ],
        out_specs=pl.BlockSpec((nch, _BN, 128), lambda i: (0, i, 0)),
        out_shape=jax.ShapeDtypeStruct((nch, n, 128), jnp.float32),
    )(x, w)


def _fuse_mid(z_raw, h, wa, wb, b2, w_next, out_c):
    nch = z_raw.shape[0]
    m_out = w_next.shape[2]
    nout = m_out // out_c

    def body(z_ref, h_ref, wa_ref, wb_ref, b_ref, w_ref, o_ref):
        h_ = h_ref[...]
        t = [_leaky(z_ref[c]) for c in range(nch)]
        a = b_ref[...] + jnp.dot(h_, wb_ref[...], preferred_element_type=jnp.float32)
        for c in range(nch):
            a = a + jnp.dot(t[c], wa_ref[c], preferred_element_type=jnp.float32)
        m = _softmax_l2(a)
        m0, m1 = m[:, 0:1], m[:, 1:2]
        acc = jnp.zeros((_BN, m_out), jnp.float32)
        for c in range(nch):
            f = m0 * t[c] + m1 * h_[:, c * 128:(c + 1) * 128]
            acc = acc + jnp.dot(f, w_ref[c], preferred_element_type=jnp.float32)
        for c in range(nout):
            o_ref[c] = acc[:, c * out_c:(c + 1) * out_c]

    return pl.pallas_call(
        body,
        grid=(N // _BN,),
        in_specs=[
            pl.BlockSpec((nch, _BN, 128), lambda i: (0, i, 0)),
            pl.BlockSpec((_BN, 128 * nch), lambda i: (i, 0)),
            pl.BlockSpec(wa.shape, lambda i: (0, 0, 0)),
            pl.BlockSpec(wb.shape, lambda i: (0, 0)),
            pl.BlockSpec((1, 2), lambda i: (0, 0)),
            pl.BlockSpec(w_next.shape, lambda i: (0, 0, 0)),
        ],
        out_specs=pl.BlockSpec((nout, _BN, out_c), lambda i: (0, i, 0)),
        out_shape=jax.ShapeDtypeStruct((nout, N, out_c), jnp.float32),
    )(z_raw, h, wa, wb, b2, w_next)


def _fuse_final(z1r, z2r, z3r, zz, wl1, wl2, wl3, wlz, blr, wz1, wz2, wz3, wzz):
    def body(z1_ref, z2_ref, z3_ref, z_ref, wl1_ref, wl2_ref, wl3_ref, wlz_ref,
             bl_ref, wz1_ref, wz2_ref, wz3_ref, wzz_ref, o_ref):
        t1 = [_leaky(z1_ref[c]) for c in range(4)]
        t2 = [_leaky(z2_ref[c]) for c in range(2)]
        t3 = [_leaky(z3_ref[c]) for c in range(2)]
        z_ = z_ref[...]
        a = bl_ref[...] + jnp.dot(z_, wlz_ref[...], preferred_element_type=jnp.float32)
        for c in range(4):
            a = a + jnp.dot(t1[c], wl1_ref[c], preferred_element_type=jnp.float32)
        for c in range(2):
            a = a + jnp.dot(t2[c], wl2_ref[c], preferred_element_type=jnp.float32)
            a = a + jnp.dot(t3[c], wl3_ref[c], preferred_element_type=jnp.float32)
        u = _softmax_l2(a)
        s1 = jnp.zeros((_BN, 16), jnp.float32)
        for c in range(4):
            s1 = s1 + jnp.dot(t1[c], wz1_ref[c], preferred_element_type=jnp.float32)
        s2 = jnp.zeros((_BN, 16), jnp.float32)
        s3 = jnp.zeros((_BN, 16), jnp.float32)
        for c in range(2):
            s2 = s2 + jnp.dot(t2[c], wz2_ref[c], preferred_element_type=jnp.float32)
            s3 = s3 + jnp.dot(t3[c], wz3_ref[c], preferred_element_type=jnp.float32)
        sz = jnp.dot(z_, wzz_ref[...], preferred_element_type=jnp.float32)
        o_ref[...] = (u[:, 0:1] * s1 + u[:, 1:2] * s2 + u[:, 2:3] * s3
                      + u[:, 3:4] * sz)

    return pl.pallas_call(
        body,
        grid=(N // _BN,),
        in_specs=[
            pl.BlockSpec((4, _BN, 128), lambda i: (0, i, 0)),
            pl.BlockSpec((2, _BN, 128), lambda i: (0, i, 0)),
            pl.BlockSpec((2, _BN, 32), lambda i: (0, i, 0)),
            pl.BlockSpec((_BN, 64), lambda i: (i, 0)),
            pl.BlockSpec(wl1.shape, lambda i: (0, 0, 0)),
            pl.BlockSpec(wl2.shape, lambda i: (0, 0, 0)),
            pl.BlockSpec(wl3.shape, lambda i: (0, 0, 0)),
            pl.BlockSpec(wlz.shape, lambda i: (0, 0)),
            pl.BlockSpec((1, 4), lambda i: (0, 0)),
            pl.BlockSpec(wz1.shape, lambda i: (0, 0, 0)),
            pl.BlockSpec(wz2.shape, lambda i: (0, 0, 0)),
            pl.BlockSpec(wz3.shape, lambda i: (0, 0, 0)),
            pl.BlockSpec(wzz.shape, lambda i: (0, 0)),
        ],
        out_specs=pl.BlockSpec((_BN, 16), lambda i: (i, 0)),
        out_shape=jax.ShapeDtypeStruct((N, 16), jnp.float32),
    )(z1r, z2r, z3r, zz, wl1, wl2, wl3, wlz, blr, wz1, wz2, wz3, wzz)


def _finish(p):
    def body(p_ref, o1_ref, o2_ref):
        net = (p_ref[0] + p_ref[1])[:, :10]
        o1_ref[...] = net
        v = net - jnp.max(net, axis=1, keepdims=True)
        e = jnp.exp(v)
        o2_ref[...] = e / jnp.sum(e, axis=1, keepdims=True)

    outs = pl.pallas_call(
        body,
        grid=(N // _BN,),
        in_specs=[pl.BlockSpec((2, _BN, 16), lambda i: (0, i, 0))],
        out_specs=[
            pl.BlockSpec((_BN, 10), lambda i: (i, 0)),
            pl.BlockSpec((_BN, 10), lambda i: (i, 0)),
        ],
        out_shape=[
            jax.ShapeDtypeStruct((N, 10), jnp.float32),
            jax.ShapeDtypeStruct((N, 10), jnp.float32),
        ],
    )(p)
    return outs[0], outs[1]


def _make_spmm(nch, C, rounds, split_edges):
    ntasks = _NTILES * (2 if split_edges else 1)
    e_per_tile = E // ntasks
    tiled = C == 128
    if tiled:
        chunk = 100
        seg_e = 5000
        nbuf = 3
    else:
        chunk = 80 if nch > 1 else 125
        seg_e = e_per_tile
        nbuf = 6
    nseg = e_per_tile // seg_e
    iters = seg_e // chunk
    n_out = 2 if split_edges else 2 * rounds
    rmain = (N // _NTILES) // 8 * 8
    rtail = N - (_NTILES - 1) * rmain
    pieces = [j * 16 for j in range(chunk // 16)]
    if chunk % 16:
        pieces.append(chunk - 16)
    mesh = plsc.VectorSubcoreMesh(core_axis_name="c", subcore_axis_name="s")

    if tiled:
        idx_scratch = [
            pltpu.VMEM((seg_e,), jnp.int32),
            pltpu.VMEM((seg_e,), jnp.int32),
            pltpu.VMEM((chunk,), jnp.int32),
        ]
    else:
        idx_scratch = [
            pltpu.VMEM((iters, chunk), jnp.int32),
            pltpu.VMEM((iters, chunk), jnp.int32),
        ]

    @functools.partial(
        pl.kernel,
        out_type=jax.ShapeDtypeStruct((n_out, N, C), jnp.float32),
        mesh=mesh,
        compiler_params=pltpu.CompilerParams(use_tc_tiling_on_sc=tiled),
        scratch_types=(
            idx_scratch
            + [pltpu.VMEM((chunk,), jnp.int32) for _ in range(nbuf)]
            + [pltpu.VMEM((chunk, C), jnp.float32) for _ in range(nbuf)]
            + [pltpu.VMEM_SHARED((N, C), jnp.float32)]
            + [pltpu.SemaphoreType.DMA for _ in range(nbuf)]
        ),
    )
    def spmm(sup_hbm, src_hbm, dst_hbm, zeros_hbm, out_hbm, *bufs):
        ni = len(idx_scratch)
        src_all, dst_all = bufs[0], bufs[1]
        dstb = bufs[2] if tiled else None
        srcx = bufs[ni:ni + nbuf]
        rows = bufs[ni + nbuf:ni + 2 * nbuf]
        acc = bufs[ni + 2 * nbuf]
        semg = bufs[ni + 2 * nbuf + 1:]
        core = lax.axis_index("c")
        sid = lax.axis_index("s")
        if tiled:
            ebase = sid * e_per_tile
        else:
            tid = core * _NTILES + sid if split_edges else sid
        rbase = sid * rmain
        last = sid == _NTILES - 1
        tbase = rmain * _NTILES

        if not tiled:
            pltpu.sync_copy(src_hbm.at[tid], src_all)
            pltpu.sync_copy(dst_hbm.at[tid], dst_all)

        for r in range(rounds):
            ch = 2 * r + core
            row_off = ch * N

            def gather_start(i, p):
                if nch > 1:
                    for j in pieces:
                        if tiled:
                            srcx[p][pl.ds(j, 16)] = (
                                src_all[pl.ds(i * chunk + j, 16)] + row_off)
                        else:
                            srcx[p][pl.ds(j, 16)] = src_all[i, pl.ds(j, 16)] + row_off
                    pltpu.async_copy(sup_hbm.at[srcx[p]], rows[p], semg[p])
                else:
                    pltpu.async_copy(sup_hbm.at[src_all.at[i]], rows[p], semg[p])

            def gather_wait(i, p):
                idx = srcx[p] if nch > 1 else src_all.at[i]
                pltpu.make_async_copy(sup_hbm.at[idx], rows[p], semg[p]).wait()

            def scatter(i, p):
                if tiled:
                    for j in pieces:
                        dstb[pl.ds(j, 16)] = dst_all[pl.ds(i * chunk + j, 16)]
                    pltpu.sync_copy(rows[p], acc.at[dstb], add=True)
                else:
                    pltpu.sync_copy(rows[p], acc.at[dst_all.at[i]], add=True)

            for s in range(nseg):
                if tiled:
                    off = ebase + s * seg_e
                    pltpu.sync_copy(src_hbm.at[pl.ds(off, seg_e)], src_all)
                    pltpu.sync_copy(dst_hbm.at[pl.ds(off, seg_e)], dst_all)
                for p in range(nbuf):
                    gather_start(p, p)
                if s == 0:
                    pltpu.sync_copy(zeros_hbm.at[pl.ds(rbase, rmain)],
                                    acc.at[pl.ds(rbase, rmain)])

                    @pl.when(last)
                    def _():
                        pltpu.sync_copy(zeros_hbm.at[pl.ds(tbase, rtail - rmain)],
                                        acc.at[pl.ds(tbase, rtail - rmain)])

                    plsc.subcore_barrier()

                def body(blk, carry):
                    for b in range(nbuf):
                        i = blk * nbuf + b
                        gather_wait(i, b)
                        k = i + nbuf - 1
                        pb = (b - 1) % nbuf

                        @pl.when((i >= 1) & (k < iters))
                        def _():
                            gather_start(k, pb)

                        scatter(i, b)

                    return carry

                lax.fori_loop(0, iters // nbuf, body, 0)
                for i in range((iters // nbuf) * nbuf, iters):
                    gather_wait(i, i % nbuf)
                    scatter(i, i % nbuf)

            plsc.subcore_barrier()
            oc = core if split_edges else ch
            pltpu.sync_copy(acc.at[pl.ds(rbase, rmain)],
                            out_hbm.at[oc, pl.ds(rbase, rmain)])

            @pl.when(last)
            def _():
                pltpu.sync_copy(acc.at[pl.ds(tbase, rtail - rmain)],
                                out_hbm.at[oc, pl.ds(tbase, rtail - rmain)])

    return spmm


_SPMM1 = _make_spmm(4, 128, 2, False)
_SPMM2 = _make_spmm(2, 128, 1, False)
_SPMM3 = _make_spmm(2, 32, 1, False)
_SPMM4 = _make_spmm(1, 16, 1, True)


def kernel(x, h1, h2, z, edge_index, W0, W1, W2, Wz, Wl, bl, Wm1, bm1, Wm2, bm2):
    dst = edge_index[0]
    src = edge_index[1]
    src80 = src.reshape(16, E // (16 * 80), 80)
    dst80 = dst.reshape(16, E // (16 * 80), 80)
    src40 = src.reshape(32, E // (32 * 125), 125)
    dst40 = dst.reshape(32, E // (32 * 125), 125)
    srcf = src.reshape(E)
    dstf = dst.reshape(E)
    z128 = jnp.zeros((N, 128), jnp.float32)
    z32 = jnp.zeros((N, 32), jnp.float32)
    z16 = jnp.zeros((N, 16), jnp.float32)

    sup1 = _matmul(x, W0)
    z1r = _SPMM1(sup1.reshape(N * 4, 128), srcf, dstf, z128)

    sup2 = _fuse_mid(z1r, h1, Wm1[:512].reshape(4, 128, 2), Wm1[512:],
                     bm1.reshape(1, 2), W1.reshape(4, 128, 256), 128)
    z2r = _SPMM2(sup2.reshape(N * 2, 128), srcf, dstf, z128)

    sup3 = _fuse_mid(z2r, h2, Wm2[:256].reshape(2, 128, 2), Wm2[256:],
                     bm2.reshape(1, 2), W2.reshape(2, 128, 64), 32)
    z3r = _SPMM3(sup3.reshape(N * 2, 32), src80, dst80, z32)

    Wzp = jnp.pad(Wz, ((0, 0), (0, 6)))
    sup4 = _fuse_final(
        z1r, z2r, z3r, z,
        Wl[:512].reshape(4, 128, 4), Wl[512:768].reshape(2, 128, 4),
        Wl[768:832].reshape(2, 32, 4), Wl[832:], bl.reshape(1, 4),
        Wzp[:512].reshape(4, 128, 16), Wzp[512:768].reshape(2, 128, 16),
        Wzp[768:832].reshape(2, 32, 16), Wzp[832:])
    p = _SPMM4(sup4, src40, dst40, z16)
    return _finish(p)
